# baseline (device time: 214588 ns/iter reference)
import functools

import jax
import jax.numpy as jnp
from jax import lax
from jax.experimental import pallas as pl
from jax.experimental.pallas import tpu as pltpu

N_DEV = 4
SQ = 2048
SKV = 2048
HQ_LOCAL = 8
DH = 128
DMODEL = 1024
QBLK = 512
N_QB = SQ // QBLK
CHUNK = SQ // N_DEV
SCALE = 0.08838834764831843
LOCAL_WINDOW = 128
GLOBAL_TOKENS = 32


def _chunk_rows(c):
    return pl.ds(c * CHUNK, CHUNK)


def _body(x_ref, wq_ref, k_ref, v_ref, wo_ref, out_ref,
          partial_ref, rs_recv_ref,
          rs_send_sems, rs_recv_sems, ag_send_sems, ag_recv_sems):
    j = pl.program_id(0)
    h = pl.program_id(1)
    my = lax.axis_index("i")
    left = lax.rem(my + N_DEV - 1, N_DEV)
    right = lax.rem(my + 1, N_DEV)
    qb = lax.rem(my - j + 2 * N_DEV, N_DEV)
    rows = _chunk_rows(qb)

    q = lax.dot_general(
        x_ref[rows, :].astype(jnp.bfloat16), wq_ref[...].astype(jnp.bfloat16),
        (((1,), (0,)), ((), ())), preferred_element_type=jnp.float32)
    k = k_ref[0].astype(jnp.bfloat16)
    v = v_ref[0].astype(jnp.bfloat16)
    s = lax.dot_general(
        q.astype(jnp.bfloat16), k, (((1,), (1,)), ((), ())),
        preferred_element_type=jnp.float32) * SCALE

    qi = qb * QBLK + lax.broadcasted_iota(jnp.int32, (QBLK, SKV), 0)
    ki = lax.broadcasted_iota(jnp.int32, (QBLK, SKV), 1)
    mask = (jnp.abs(qi - ki) <= LOCAL_WINDOW) | (ki < GLOBAL_TOKENS) | (
        qi < GLOBAL_TOKENS)
    s = jnp.where(mask, s, -1e9)

    m = jnp.max(s, axis=1, keepdims=True)
    e = jnp.exp(s - m)
    w = e / jnp.sum(e, axis=1, keepdims=True)

    ctx = lax.dot_general(
        w.astype(jnp.bfloat16), v, (((1,), (0,)), ((), ())),
        preferred_element_type=jnp.float32)
    contrib = lax.dot_general(
        ctx.astype(jnp.bfloat16), wo_ref[...].astype(jnp.bfloat16),
        (((1,), (0,)), ((), ())), preferred_element_type=jnp.float32)

    @pl.when(h == 0)
    def _():
        partial_ref[rows, :] = contrib

    @pl.when(h != 0)
    def _():
        partial_ref[rows, :] = partial_ref[rows, :] + contrib

    def rs_send(step):
        c = lax.rem(my - step + 2 * N_DEV, N_DEV)
        rdma = pltpu.make_async_remote_copy(
            src_ref=partial_ref.at[_chunk_rows(c)],
            dst_ref=rs_recv_ref.at[step],
            send_sem=rs_send_sems.at[step],
            recv_sem=rs_recv_sems.at[step],
            device_id=(right,),
            device_id_type=pl.DeviceIdType.MESH,
        )
        rdma.start()
        return rdma

    def rs_wait_and_add(step):
        c = lax.rem(my - step - 1 + 2 * N_DEV, N_DEV)
        rdma = pltpu.make_async_remote_copy(
            src_ref=partial_ref.at[_chunk_rows(c)],
            dst_ref=rs_recv_ref.at[step],
            send_sem=rs_send_sems.at[step],
            recv_sem=rs_recv_sems.at[step],
            device_id=(right,),
            device_id_type=pl.DeviceIdType.MESH,
        )
        rdma.wait_recv()
        rrows = _chunk_rows(c)
        partial_ref[rrows, :] = partial_ref[rrows, :] + rs_recv_ref[step]

    def rs_wait_send(step):
        c = lax.rem(my - step + 2 * N_DEV, N_DEV)
        rdma = pltpu.make_async_remote_copy(
            src_ref=partial_ref.at[_chunk_rows(c)],
            dst_ref=rs_recv_ref.at[step],
            send_sem=rs_send_sems.at[step],
            recv_sem=rs_recv_sems.at[step],
            device_id=(right,),
            device_id_type=pl.DeviceIdType.MESH,
        )
        rdma.wait_send()

    def ag_copy(c, sem_idx, target):
        return pltpu.make_async_remote_copy(
            src_ref=out_ref.at[_chunk_rows(c)],
            dst_ref=out_ref.at[_chunk_rows(c)],
            send_sem=ag_send_sems.at[sem_idx],
            recv_sem=ag_recv_sems.at[sem_idx],
            device_id=(target,),
            device_id_type=pl.DeviceIdType.MESH,
        )

    last_h = h == HQ_LOCAL - 1

    @pl.when(jnp.logical_and(last_h, j == 0))
    def _():
        barrier_sem = pltpu.get_barrier_semaphore()
        for nbr in (left, right):
            pl.semaphore_signal(barrier_sem, inc=1, device_id=(nbr,),
                                device_id_type=pl.DeviceIdType.MESH)
        pl.semaphore_wait(barrier_sem, 2)
        rs_send(0)

    @pl.when(jnp.logical_and(last_h, jnp.logical_and(j > 0, j < N_QB - 1)))
    def _():
        for step in range(N_QB - 2):
            @pl.when(j == step + 1)
            def _():
                rs_wait_and_add(step)
                rs_wait_send(step)
                rs_send(step + 1)

    @pl.when(jnp.logical_and(last_h, j == N_QB - 1))
    def _():
        rs_wait_and_add(N_DEV - 2)
        rs_wait_send(N_DEV - 2)

        c_own = lax.rem(my + 1, N_DEV)
        orows = _chunk_rows(c_own)
        out_ref[orows, :] = partial_ref[orows, :]

        send_r = ag_copy(c_own, 0, right)
        send_l = ag_copy(c_own, 1, left)
        send_r.start()
        send_l.start()

        ag_copy(my, 0, right).wait_recv()
        fwd = ag_copy(my, 2, right)
        fwd.start()

        ag_copy(lax.rem(my + 2, N_DEV), 1, right).wait_recv()
        ag_copy(left, 2, right).wait_recv()

        send_r.wait_send()
        send_l.wait_send()
        fwd.wait_send()

        @functools.partial(pl.run_scoped,
                           second_barrier=pltpu.SemaphoreType.REGULAR)
        def _(second_barrier):
            for nbr in (left, right):
                pl.semaphore_signal(second_barrier, inc=1, device_id=(nbr,),
                                    device_id_type=pl.DeviceIdType.MESH)
            pl.semaphore_wait(second_barrier, 2)


def kernel(x, Wq, K_ext, V_ext, Wo):
    my = lax.axis_index("i")
    x2 = x.reshape(SQ, DMODEL)
    wq_s = lax.dynamic_slice(Wq, (0, my * HQ_LOCAL * DH), (DMODEL, HQ_LOCAL * DH))
    wo_s = lax.dynamic_slice(Wo, (my * HQ_LOCAL * DH, 0), (HQ_LOCAL * DH, DMODEL))
    k = jnp.transpose(K_ext.reshape(SKV, HQ_LOCAL, DH), (1, 0, 2))
    v = jnp.transpose(V_ext.reshape(SKV, HQ_LOCAL, DH), (1, 0, 2))

    out = pl.pallas_call(
        _body,
        grid=(N_QB, HQ_LOCAL),
        in_specs=[
            pl.BlockSpec((SQ, DMODEL), lambda j, h: (0, 0)),
            pl.BlockSpec((DMODEL, DH), lambda j, h: (0, h)),
            pl.BlockSpec((1, SKV, DH), lambda j, h: (h, 0, 0)),
            pl.BlockSpec((1, SKV, DH), lambda j, h: (h, 0, 0)),
            pl.BlockSpec((DH, DMODEL), lambda j, h: (h, 0)),
        ],
        out_specs=pl.BlockSpec((SQ, DMODEL), lambda j, h: (0, 0)),
        out_shape=jax.ShapeDtypeStruct((SQ, DMODEL), jnp.float32),
        scratch_shapes=[
            pltpu.VMEM((SQ, DMODEL), jnp.float32),
            pltpu.VMEM((N_DEV - 1, CHUNK, DMODEL), jnp.float32),
            pltpu.SemaphoreType.DMA((N_DEV - 1,)),
            pltpu.SemaphoreType.DMA((N_DEV - 1,)),
            pltpu.SemaphoreType.DMA((N_DEV - 1,)),
            pltpu.SemaphoreType.DMA((N_DEV - 1,)),
        ],
        compiler_params=pltpu.CompilerParams(
            collective_id=0,
            dimension_semantics=("arbitrary", "arbitrary"),
        ),
    )(x2, wq_s, k, v, wo_s)
    return out.reshape(1, SQ, DMODEL)


# device time: 162211 ns/iter; 1.3229x vs baseline; 1.3229x over previous
import functools

import jax
import jax.numpy as jnp
from jax import lax
from jax.experimental import pallas as pl
from jax.experimental.pallas import tpu as pltpu

import os
_COMM = os.environ.get("KERNEL_NO_COMM", "0") != "1"

N_DEV = 4
SQ = 2048
SKV = 2048
HQ_LOCAL = 8
DH = 128
DMODEL = 1024
QBLK = 512
N_QB = SQ // QBLK
CHUNK = SQ // N_DEV
SCALE = 0.08838834764831843
LOCAL_WINDOW = 128
GLOBAL_TOKENS = 32


def _chunk_rows(c):
    return pl.ds(c * CHUNK, CHUNK)


def _body(x_ref, wq_ref, k_ref, v_ref, wo_ref, out_ref,
          partial_ref, rs_recv_ref,
          rs_send_sems, rs_recv_sems, ag_send_sems, ag_recv_sems):
    j = pl.program_id(0)
    h = pl.program_id(1)
    my = lax.axis_index("i")
    left = lax.rem(my + N_DEV - 1, N_DEV)
    right = lax.rem(my + 1, N_DEV)
    qb = lax.rem(my - j + 2 * N_DEV, N_DEV)
    rows = _chunk_rows(qb)

    q = lax.dot_general(
        x_ref[rows, :].astype(jnp.bfloat16), wq_ref[...].astype(jnp.bfloat16),
        (((1,), (0,)), ((), ())), preferred_element_type=jnp.float32)
    k = k_ref[0].astype(jnp.bfloat16)
    v = v_ref[0].astype(jnp.bfloat16)
    s = lax.dot_general(
        q.astype(jnp.bfloat16), k, (((1,), (1,)), ((), ())),
        preferred_element_type=jnp.float32) * SCALE

    qi = qb * QBLK + lax.broadcasted_iota(jnp.int32, (QBLK, SKV), 0)
    ki = lax.broadcasted_iota(jnp.int32, (QBLK, SKV), 1)
    mask = (jnp.abs(qi - ki) <= LOCAL_WINDOW) | (ki < GLOBAL_TOKENS) | (
        qi < GLOBAL_TOKENS)
    s = jnp.where(mask, s, -1e9)

    m = jnp.max(s, axis=1, keepdims=True)
    e = jnp.exp(s - m)
    w = e / jnp.sum(e, axis=1, keepdims=True)

    ctx = lax.dot_general(
        w.astype(jnp.bfloat16), v, (((1,), (0,)), ((), ())),
        preferred_element_type=jnp.float32)
    contrib = lax.dot_general(
        ctx.astype(jnp.bfloat16), wo_ref[...].astype(jnp.bfloat16),
        (((1,), (0,)), ((), ())), preferred_element_type=jnp.float32)

    @pl.when(h == 0)
    def _():
        partial_ref[rows, :] = contrib

    @pl.when(h != 0)
    def _():
        partial_ref[rows, :] = partial_ref[rows, :] + contrib

    def rs_send(step):
        c = lax.rem(my - step + 2 * N_DEV, N_DEV)
        rdma = pltpu.make_async_remote_copy(
            src_ref=partial_ref.at[_chunk_rows(c)],
            dst_ref=rs_recv_ref.at[step],
            send_sem=rs_send_sems.at[step],
            recv_sem=rs_recv_sems.at[step],
            device_id=(right,),
            device_id_type=pl.DeviceIdType.MESH,
        )
        rdma.start()
        return rdma

    def rs_wait_and_add(step):
        c = lax.rem(my - step - 1 + 2 * N_DEV, N_DEV)
        rdma = pltpu.make_async_remote_copy(
            src_ref=partial_ref.at[_chunk_rows(c)],
            dst_ref=rs_recv_ref.at[step],
            send_sem=rs_send_sems.at[step],
            recv_sem=rs_recv_sems.at[step],
            device_id=(right,),
            device_id_type=pl.DeviceIdType.MESH,
        )
        rdma.wait_recv()
        rrows = _chunk_rows(c)
        partial_ref[rrows, :] = partial_ref[rrows, :] + rs_recv_ref[step]

    def rs_wait_send(step):
        c = lax.rem(my - step + 2 * N_DEV, N_DEV)
        rdma = pltpu.make_async_remote_copy(
            src_ref=partial_ref.at[_chunk_rows(c)],
            dst_ref=rs_recv_ref.at[step],
            send_sem=rs_send_sems.at[step],
            recv_sem=rs_recv_sems.at[step],
            device_id=(right,),
            device_id_type=pl.DeviceIdType.MESH,
        )
        rdma.wait_send()

    def ag_copy(c, sem_idx, target):
        return pltpu.make_async_remote_copy(
            src_ref=out_ref.at[_chunk_rows(c)],
            dst_ref=out_ref.at[_chunk_rows(c)],
            send_sem=ag_send_sems.at[sem_idx],
            recv_sem=ag_recv_sems.at[sem_idx],
            device_id=(target,),
            device_id_type=pl.DeviceIdType.MESH,
        )

    last_h = (h == HQ_LOCAL - 1) if _COMM else jnp.bool_(False)

    if not _COMM:
        @pl.when(jnp.logical_and(h == HQ_LOCAL - 1, j == N_QB - 1))
        def _():
            out_ref[...] = partial_ref[...]

    @pl.when(jnp.logical_and(last_h, j == 0))
    def _():
        barrier_sem = pltpu.get_barrier_semaphore()
        for nbr in (left, right):
            pl.semaphore_signal(barrier_sem, inc=1, device_id=(nbr,),
                                device_id_type=pl.DeviceIdType.MESH)
        pl.semaphore_wait(barrier_sem, 2)
        rs_send(0)

    @pl.when(jnp.logical_and(last_h, jnp.logical_and(j > 0, j < N_QB - 1)))
    def _():
        for step in range(N_QB - 2):
            @pl.when(j == step + 1)
            def _():
                rs_wait_and_add(step)
                rs_wait_send(step)
                rs_send(step + 1)

    @pl.when(jnp.logical_and(last_h, j == N_QB - 1))
    def _():
        rs_wait_and_add(N_DEV - 2)
        rs_wait_send(N_DEV - 2)

        c_own = lax.rem(my + 1, N_DEV)
        orows = _chunk_rows(c_own)
        out_ref[orows, :] = partial_ref[orows, :]

        send_r = ag_copy(c_own, 0, right)
        send_l = ag_copy(c_own, 1, left)
        send_r.start()
        send_l.start()

        ag_copy(my, 0, right).wait_recv()
        fwd = ag_copy(my, 2, right)
        fwd.start()

        ag_copy(lax.rem(my + 2, N_DEV), 1, right).wait_recv()
        ag_copy(left, 2, right).wait_recv()

        send_r.wait_send()
        send_l.wait_send()
        fwd.wait_send()

        @functools.partial(pl.run_scoped,
                           second_barrier=pltpu.SemaphoreType.REGULAR)
        def _(second_barrier):
            for nbr in (left, right):
                pl.semaphore_signal(second_barrier, inc=1, device_id=(nbr,),
                                    device_id_type=pl.DeviceIdType.MESH)
            pl.semaphore_wait(second_barrier, 2)


def kernel(x, Wq, K_ext, V_ext, Wo):
    my = lax.axis_index("i")
    x2 = x.reshape(SQ, DMODEL)
    wq_s = lax.dynamic_slice(Wq, (0, my * HQ_LOCAL * DH), (DMODEL, HQ_LOCAL * DH))
    wo_s = lax.dynamic_slice(Wo, (my * HQ_LOCAL * DH, 0), (HQ_LOCAL * DH, DMODEL))
    k = jnp.transpose(K_ext.reshape(SKV, HQ_LOCAL, DH), (1, 0, 2))
    v = jnp.transpose(V_ext.reshape(SKV, HQ_LOCAL, DH), (1, 0, 2))

    out = pl.pallas_call(
        _body,
        grid=(N_QB, HQ_LOCAL),
        in_specs=[
            pl.BlockSpec((SQ, DMODEL), lambda j, h: (0, 0)),
            pl.BlockSpec((DMODEL, DH), lambda j, h: (0, h)),
            pl.BlockSpec((1, SKV, DH), lambda j, h: (h, 0, 0)),
            pl.BlockSpec((1, SKV, DH), lambda j, h: (h, 0, 0)),
            pl.BlockSpec((DH, DMODEL), lambda j, h: (h, 0)),
        ],
        out_specs=pl.BlockSpec((SQ, DMODEL), lambda j, h: (0, 0)),
        out_shape=jax.ShapeDtypeStruct((SQ, DMODEL), jnp.float32),
        scratch_shapes=[
            pltpu.VMEM((SQ, DMODEL), jnp.float32),
            pltpu.VMEM((N_DEV - 1, CHUNK, DMODEL), jnp.float32),
            pltpu.SemaphoreType.DMA((N_DEV - 1,)),
            pltpu.SemaphoreType.DMA((N_DEV - 1,)),
            pltpu.SemaphoreType.DMA((N_DEV - 1,)),
            pltpu.SemaphoreType.DMA((N_DEV - 1,)),
        ],
        compiler_params=pltpu.CompilerParams(
            collective_id=0,
            dimension_semantics=("arbitrary", "arbitrary"),
        ),
    )(x2, wq_s, k, v, wo_s)
    return out.reshape(1, SQ, DMODEL)


# device time: 121358 ns/iter; 1.7682x vs baseline; 1.3366x over previous
import functools

import jax
import jax.numpy as jnp
from jax import lax
from jax.experimental import pallas as pl
from jax.experimental.pallas import tpu as pltpu

import os
_COMM = os.environ.get("KERNEL_NO_COMM", "0") != "1"

N_DEV = 4
SQ = 2048
SKV = 2048
HQ_LOCAL = 8
DH = 128
DMODEL = 1024
QBLK = 512
N_QB = SQ // QBLK
CHUNK = SQ // N_DEV
SCALE = 0.08838834764831843
LOCAL_WINDOW = 128
GLOBAL_TOKENS = 32
GBLK = 128
WWIN = 768


def _chunk_rows(c):
    return pl.ds(c * CHUNK, CHUNK)


def _body(x_ref, wq_ref, k_ref, v_ref, wo_ref, out_ref,
          partial_ref, rs_recv_ref,
          rs_send_sems, rs_recv_sems, ag_send_sems, ag_recv_sems):
    j = pl.program_id(0)
    h = pl.program_id(1)
    my = lax.axis_index("i")
    left = lax.rem(my + N_DEV - 1, N_DEV)
    right = lax.rem(my + 1, N_DEV)
    qb = lax.rem(my - j + 2 * N_DEV, N_DEV)
    rows = _chunk_rows(qb)

    q = lax.dot_general(
        x_ref[rows, :], wq_ref[...],
        (((1,), (0,)), ((), ())), preferred_element_type=jnp.float32
    ).astype(jnp.bfloat16)

    def accum(contrib):
        @pl.when(h == 0)
        def _():
            partial_ref[rows, :] = contrib

        @pl.when(h != 0)
        def _():
            partial_ref[rows, :] = partial_ref[rows, :] + contrib

    def finish(parts):
        m = parts[0][0].max(axis=1, keepdims=True)
        for s, _ in parts[1:]:
            m = jnp.maximum(m, s.max(axis=1, keepdims=True))
        es = [jnp.exp(s - m) for s, _ in parts]
        denom = es[0].sum(axis=1, keepdims=True)
        for e in es[1:]:
            denom = denom + e.sum(axis=1, keepdims=True)
        ctx = None
        for e, (_, vp) in zip(es, parts):
            w = (e / denom).astype(jnp.bfloat16)
            c = lax.dot_general(w, vp, (((1,), (0,)), ((), ())),
                                preferred_element_type=jnp.float32)
            ctx = c if ctx is None else ctx + c
        contrib = lax.dot_general(
            ctx.astype(jnp.bfloat16), wo_ref[...], (((1,), (0,)), ((), ())),
            preferred_element_type=jnp.float32)
        accum(contrib)

    @pl.when(qb == 0)
    def _():
        s = lax.dot_general(
            q, k_ref[0], (((1,), (1,)), ((), ())),
            preferred_element_type=jnp.float32) * SCALE
        qi = lax.broadcasted_iota(jnp.int32, (QBLK, SKV), 0)
        ki = lax.broadcasted_iota(jnp.int32, (QBLK, SKV), 1)
        mask = (jnp.abs(qi - ki) <= LOCAL_WINDOW) | (ki < GLOBAL_TOKENS) | (
            qi < GLOBAL_TOKENS)
        s = jnp.where(mask, s, -1e9)
        finish([(s, v_ref[0])])

    @pl.when(qb != 0)
    def _():
        w0 = jnp.minimum(qb * QBLK - LOCAL_WINDOW, SKV - WWIN)
        s_g = lax.dot_general(
            q, k_ref[0, 0:GBLK, :], (((1,), (1,)), ((), ())),
            preferred_element_type=jnp.float32) * SCALE
        ki_g = lax.broadcasted_iota(jnp.int32, (QBLK, GBLK), 1)
        s_g = jnp.where(ki_g < GLOBAL_TOKENS, s_g, -1e9)

        s_w = lax.dot_general(
            q, k_ref[0, pl.ds(w0, WWIN), :], (((1,), (1,)), ((), ())),
            preferred_element_type=jnp.float32) * SCALE
        qi = qb * QBLK + lax.broadcasted_iota(jnp.int32, (QBLK, WWIN), 0)
        ki_w = w0 + lax.broadcasted_iota(jnp.int32, (QBLK, WWIN), 1)
        s_w = jnp.where(jnp.abs(qi - ki_w) <= LOCAL_WINDOW, s_w, -1e9)

        finish([(s_g, v_ref[0, 0:GBLK, :]),
                (s_w, v_ref[0, pl.ds(w0, WWIN), :])])

    def rs_send(step):
        c = lax.rem(my - step + 2 * N_DEV, N_DEV)
        rdma = pltpu.make_async_remote_copy(
            src_ref=partial_ref.at[_chunk_rows(c)],
            dst_ref=rs_recv_ref.at[step],
            send_sem=rs_send_sems.at[step],
            recv_sem=rs_recv_sems.at[step],
            device_id=(right,),
            device_id_type=pl.DeviceIdType.MESH,
        )
        rdma.start()
        return rdma

    def rs_wait_and_add(step):
        c = lax.rem(my - step - 1 + 2 * N_DEV, N_DEV)
        rdma = pltpu.make_async_remote_copy(
            src_ref=partial_ref.at[_chunk_rows(c)],
            dst_ref=rs_recv_ref.at[step],
            send_sem=rs_send_sems.at[step],
            recv_sem=rs_recv_sems.at[step],
            device_id=(right,),
            device_id_type=pl.DeviceIdType.MESH,
        )
        rdma.wait_recv()
        rrows = _chunk_rows(c)
        partial_ref[rrows, :] = partial_ref[rrows, :] + rs_recv_ref[step]

    def rs_wait_send(step):
        c = lax.rem(my - step + 2 * N_DEV, N_DEV)
        rdma = pltpu.make_async_remote_copy(
            src_ref=partial_ref.at[_chunk_rows(c)],
            dst_ref=rs_recv_ref.at[step],
            send_sem=rs_send_sems.at[step],
            recv_sem=rs_recv_sems.at[step],
            device_id=(right,),
            device_id_type=pl.DeviceIdType.MESH,
        )
        rdma.wait_send()

    def ag_copy(c, sem_idx, target):
        return pltpu.make_async_remote_copy(
            src_ref=out_ref.at[_chunk_rows(c)],
            dst_ref=out_ref.at[_chunk_rows(c)],
            send_sem=ag_send_sems.at[sem_idx],
            recv_sem=ag_recv_sems.at[sem_idx],
            device_id=(target,),
            device_id_type=pl.DeviceIdType.MESH,
        )

    last_h = (h == HQ_LOCAL - 1) if _COMM else jnp.bool_(False)

    if not _COMM:
        @pl.when(jnp.logical_and(h == HQ_LOCAL - 1, j == N_QB - 1))
        def _():
            out_ref[...] = partial_ref[...]

    @pl.when(jnp.logical_and(last_h, j == 0))
    def _():
        barrier_sem = pltpu.get_barrier_semaphore()
        for nbr in (left, right):
            pl.semaphore_signal(barrier_sem, inc=1, device_id=(nbr,),
                                device_id_type=pl.DeviceIdType.MESH)
        pl.semaphore_wait(barrier_sem, 2)
        rs_send(0)

    @pl.when(jnp.logical_and(last_h, jnp.logical_and(j > 0, j < N_QB - 1)))
    def _():
        for step in range(N_QB - 2):
            @pl.when(j == step + 1)
            def _():
                rs_wait_and_add(step)
                rs_wait_send(step)
                rs_send(step + 1)

    @pl.when(jnp.logical_and(last_h, j == N_QB - 1))
    def _():
        rs_wait_and_add(N_DEV - 2)
        rs_wait_send(N_DEV - 2)

        c_own = lax.rem(my + 1, N_DEV)
        orows = _chunk_rows(c_own)
        out_ref[orows, :] = partial_ref[orows, :]

        send_r = ag_copy(c_own, 0, right)
        send_l = ag_copy(c_own, 1, left)
        send_r.start()
        send_l.start()

        ag_copy(my, 0, right).wait_recv()
        fwd = ag_copy(my, 2, right)
        fwd.start()

        ag_copy(lax.rem(my + 2, N_DEV), 1, right).wait_recv()
        ag_copy(left, 2, right).wait_recv()

        send_r.wait_send()
        send_l.wait_send()
        fwd.wait_send()

        @functools.partial(pl.run_scoped,
                           second_barrier=pltpu.SemaphoreType.REGULAR)
        def _(second_barrier):
            for nbr in (left, right):
                pl.semaphore_signal(second_barrier, inc=1, device_id=(nbr,),
                                    device_id_type=pl.DeviceIdType.MESH)
            pl.semaphore_wait(second_barrier, 2)


def kernel(x, Wq, K_ext, V_ext, Wo):
    my = lax.axis_index("i")
    x2 = x.reshape(SQ, DMODEL).astype(jnp.bfloat16)
    wq_s = lax.dynamic_slice(
        Wq, (0, my * HQ_LOCAL * DH), (DMODEL, HQ_LOCAL * DH)
    ).astype(jnp.bfloat16)
    wo_s = lax.dynamic_slice(
        Wo, (my * HQ_LOCAL * DH, 0), (HQ_LOCAL * DH, DMODEL)
    ).astype(jnp.bfloat16)
    k = jnp.transpose(K_ext.reshape(SKV, HQ_LOCAL, DH), (1, 0, 2)).astype(
        jnp.bfloat16)
    v = jnp.transpose(V_ext.reshape(SKV, HQ_LOCAL, DH), (1, 0, 2)).astype(
        jnp.bfloat16)

    out = pl.pallas_call(
        _body,
        grid=(N_QB, HQ_LOCAL),
        in_specs=[
            pl.BlockSpec((SQ, DMODEL), lambda j, h: (0, 0)),
            pl.BlockSpec((DMODEL, DH), lambda j, h: (0, h)),
            pl.BlockSpec((1, SKV, DH), lambda j, h: (h, 0, 0)),
            pl.BlockSpec((1, SKV, DH), lambda j, h: (h, 0, 0)),
            pl.BlockSpec((DH, DMODEL), lambda j, h: (h, 0)),
        ],
        out_specs=pl.BlockSpec((SQ, DMODEL), lambda j, h: (0, 0)),
        out_shape=jax.ShapeDtypeStruct((SQ, DMODEL), jnp.float32),
        scratch_shapes=[
            pltpu.VMEM((SQ, DMODEL), jnp.float32),
            pltpu.VMEM((N_DEV - 1, CHUNK, DMODEL), jnp.float32),
            pltpu.SemaphoreType.DMA((N_DEV - 1,)),
            pltpu.SemaphoreType.DMA((N_DEV - 1,)),
            pltpu.SemaphoreType.DMA((N_DEV - 1,)),
            pltpu.SemaphoreType.DMA((N_DEV - 1,)),
        ],
        compiler_params=pltpu.CompilerParams(
            collective_id=0,
            dimension_semantics=("arbitrary", "arbitrary"),
        ),
    )(x2, wq_s, k, v, wo_s)
    return out.reshape(1, SQ, DMODEL)
